# Initial kernel scaffold; baseline (speedup 1.0000x reference)
import functools

import jax
import jax.numpy as jnp
from jax import lax
from jax.experimental import pallas as pl
from jax.experimental.pallas import tpu as pltpu
from jax.experimental.pallas import tpu_sc as plsc

N = 10000
E = 160000
NUM_R = 5000
D_FEAT = 128
EMB = 64
LLM_DIM = 768

NC = 2
NS = 16
L = 16

CH = 128
KCH = -(-E // (NS * CH))
EP = NS * CH * KCH
RPT = N // NS


def _spmm_sc(table2, src_t, dst_t, w_t, zeros):
    mesh = plsc.VectorSubcoreMesh(core_axis_name="c", subcore_axis_name="s")

    @functools.partial(
        pl.kernel,
        out_type=jax.ShapeDtypeStruct((NC * N, EMB), jnp.float32),
        mesh=mesh,
        scratch_types=[
            pltpu.VMEM((KCH, CH), jnp.int32),
            pltpu.VMEM((KCH, CH), jnp.int32),
            pltpu.VMEM((KCH, CH), jnp.float32),
            pltpu.VMEM((CH, EMB), jnp.float32),
            pltpu.VMEM_SHARED((N, EMB), jnp.float32),
            pltpu.SemaphoreType.DMA,
        ],
    )
    def k(table_hbm, src_hbm, dst_hbm, w_hbm, z_hbm, out_hbm,
          src_v, dst_v, w_v, rows_v, acc_sh, sem):
        c = lax.axis_index("c")
        s = lax.axis_index("s")

        pltpu.sync_copy(src_hbm.at[s], src_v)
        pltpu.sync_copy(dst_hbm.at[s], dst_v)
        pltpu.sync_copy(w_hbm.at[s], w_v)

        off = (c * N).astype(jnp.int32)

        def add_off(j, carry):
            for q in range(CH // L):
                sl = pl.ds(q * L, L)
                src_v[j, sl] = src_v[j, sl] + off
            return carry

        lax.fori_loop(0, KCH, add_off, 0)

        pltpu.sync_copy(z_hbm.at[pl.ds(s * RPT, RPT)],
                        acc_sh.at[pl.ds(s * RPT, RPT)])
        plsc.subcore_barrier()

        def chunk(j, carry):
            pltpu.async_copy(table_hbm.at[src_v.at[j]], rows_v, sem).wait()

            for r in range(CH):
                wr = w_v[j, r]
                for q in range(EMB // L):
                    sl = pl.ds(q * L, L)
                    rows_v[r, sl] = rows_v[r, sl] * wr

            pltpu.sync_copy(rows_v, acc_sh.at[dst_v.at[j]], add=True)
            return carry

        lax.fori_loop(0, KCH, chunk, 0)
        plsc.subcore_barrier()

        pltpu.sync_copy(acc_sh.at[pl.ds(s * RPT, RPT)],
                        out_hbm.at[pl.ds(c * N + s * RPT, RPT)])

    return k(table2, src_t, dst_t, w_t, zeros)


_BM = 1000


def _dot(a, b):
    return lax.dot_general(a, b, (((1,), (0,)), ((), ())),
                           preferred_element_type=jnp.float32)


def _mm(h, w):
    m, kdim = h.shape

    def body(h_ref, w_ref, o_ref):
        o_ref[...] = _dot(h_ref[...], w_ref[...])

    return pl.pallas_call(
        body,
        grid=(m // _BM,),
        in_specs=[
            pl.BlockSpec((_BM, kdim), lambda i: (i, 0)),
            pl.BlockSpec((kdim, EMB), lambda i: (0, 0)),
        ],
        out_specs=pl.BlockSpec((_BM, EMB), lambda i: (i, 0)),
        out_shape=jax.ShapeDtypeStruct((m, EMB), jnp.float32),
    )(h, w)


def _elu(t):
    return jnp.where(t > 0, t, jnp.expm1(t))


def _elu_mm(t, w):
    def body(t_ref, w_ref, h_ref, g_ref):
        h = _elu(t_ref[...])
        h_ref[...] = h
        g_ref[...] = _dot(h, w_ref[...])

    return pl.pallas_call(
        body,
        grid=(N // _BM,),
        in_specs=[
            pl.BlockSpec((_BM, EMB), lambda i: (i, 0)),
            pl.BlockSpec((EMB, EMB), lambda i: (0, 0)),
        ],
        out_specs=[
            pl.BlockSpec((_BM, EMB), lambda i: (i, 0)),
            pl.BlockSpec((_BM, EMB), lambda i: (i, 0)),
        ],
        out_shape=[
            jax.ShapeDtypeStruct((N, EMB), jnp.float32),
            jax.ShapeDtypeStruct((N, EMB), jnp.float32),
        ],
    )(t, w)


def _combine(h1a, h2a, t3a, h1b, h2b, t3b, a_layer, a_drug, a_dis, wd):
    assert NUM_R % _BM == 0

    def body(h1a_ref, h2a_ref, t3a_ref, h1b_ref, h2b_ref, t3b_ref,
             al_ref, ad_ref, as_ref, wd_ref, fin_ref, llm_ref, r_ref):
        i = pl.program_id(0)
        al = al_ref[...]
        wl = jnp.exp(al - jnp.max(al))
        wl = wl / jnp.sum(wl)
        e3a = _elu(t3a_ref[...])
        e3b = _elu(t3b_ref[...])
        emb = wl[0] * h1a_ref[...] + wl[1] * h2a_ref[...] + wl[2] * e3a
        lemb = wl[0] * h1b_ref[...] + wl[1] * h2b_ref[...] + wl[2] * e3b
        llm_ref[...] = lemb

        ad = ad_ref[...]
        wd2 = jnp.exp(ad - jnp.max(ad))
        wd2 = wd2 / jnp.sum(wd2)
        asv = as_ref[...]
        ws2 = jnp.exp(asv - jnp.max(asv))
        ws2 = ws2 / jnp.sum(ws2)

        is_drug = i < NUM_R // _BM
        w0 = jnp.where(is_drug, wd2[0], ws2[0])
        w1 = jnp.where(is_drug, wd2[1], ws2[1])
        fin = w0 * emb + w1 * lemb
        fin_ref[...] = fin
        r_ref[...] = _dot(fin, wd_ref[...])

    vspec = pl.BlockSpec((_BM, EMB), lambda i: (i, 0))
    return pl.pallas_call(
        body,
        grid=(N // _BM,),
        in_specs=[vspec] * 6 + [
            pl.BlockSpec((3,), lambda i: (0,)),
            pl.BlockSpec((2,), lambda i: (0,)),
            pl.BlockSpec((2,), lambda i: (0,)),
            pl.BlockSpec((EMB, EMB), lambda i: (0, 0)),
        ],
        out_specs=[vspec, vspec, vspec],
        out_shape=[
            jax.ShapeDtypeStruct((N, EMB), jnp.float32),
            jax.ShapeDtypeStruct((N, EMB), jnp.float32),
            jax.ShapeDtypeStruct((N, EMB), jnp.float32),
        ],
    )(h1a, h2a, t3a, h1b, h2b, t3b, a_layer, a_drug, a_dis, wd)


def _decoder(r, d):
    bm = 1000

    def body(r_ref, d_ref, o_ref):
        acc = lax.dot_general(r_ref[...], d_ref[...],
                              (((1,), (1,)), ((), ())),
                              preferred_element_type=jnp.float32)
        o_ref[...] = jax.nn.sigmoid(acc)

    return pl.pallas_call(
        body,
        grid=(NUM_R // bm, NUM_R // bm),
        in_specs=[
            pl.BlockSpec((bm, EMB), lambda i, j: (i, 0)),
            pl.BlockSpec((bm, EMB), lambda i, j: (j, 0)),
        ],
        out_specs=pl.BlockSpec((bm, bm), lambda i, j: (i, j)),
        out_shape=jax.ShapeDtypeStruct((NUM_R, NUM_R), jnp.float32),
    )(r, d)


def kernel(x, drug_emb, dis_emb, edge_index, edge_weight,
           W1, W2, W3, W4, W5, W6, a_layer, a_drug, a_dis, Wd):
    pad = EP - E
    src = jnp.concatenate(
        [edge_index[0].astype(jnp.int32), jnp.zeros((pad,), jnp.int32)])
    dst = jnp.concatenate(
        [edge_index[1].astype(jnp.int32), jnp.zeros((pad,), jnp.int32)])
    wgt = jnp.concatenate(
        [edge_weight.astype(jnp.float32), jnp.zeros((pad,), jnp.float32)])
    src_t = src.reshape(NS, KCH, CH)
    dst_t = dst.reshape(NS, KCH, CH)
    w_t = wgt.reshape(NS, KCH, CH)
    zeros = jnp.zeros((N, EMB), jnp.float32)

    llm_x = jnp.concatenate([drug_emb, dis_emb], axis=0)

    g1a = _mm(x, W1)
    g1b = _mm(llm_x, W4)

    t1 = _spmm_sc(jnp.concatenate([g1a, g1b], axis=0), src_t, dst_t, w_t, zeros)
    h1a, g2a = _elu_mm(t1[:N], W2)
    h1b, g2b = _elu_mm(t1[N:], W5)

    t2 = _spmm_sc(jnp.concatenate([g2a, g2b], axis=0), src_t, dst_t, w_t, zeros)
    h2a, g3a = _elu_mm(t2[:N], W3)
    h2b, g3b = _elu_mm(t2[N:], W6)

    t3 = _spmm_sc(jnp.concatenate([g3a, g3b], axis=0), src_t, dst_t, w_t, zeros)

    final, llm_embeddings, r_full = _combine(
        h1a, h2a, t3[:N], h1b, h2b, t3[N:], a_layer, a_drug, a_dis, Wd)

    recon = _decoder(r_full[:NUM_R], final[NUM_R:]).reshape(-1)
    return (recon, final, llm_embeddings)


# speedup vs baseline: 4.9790x; 4.9790x over previous
import functools

import jax
import jax.numpy as jnp
from jax import lax
from jax.experimental import pallas as pl
from jax.experimental.pallas import tpu as pltpu
from jax.experimental.pallas import tpu_sc as plsc

N = 10000
E = 160000
NUM_R = 5000
D_FEAT = 128
EMB = 64
LLM_DIM = 768

NC = 2
NS = 16
L = 16

CH = 128
KCH = -(-E // (NS * CH))
EP = NS * CH * KCH
RPT = 632
NP = NS * RPT


def _spmm_sc(table2, src_t, dst_t, w_t, zeros):
    mesh = plsc.VectorSubcoreMesh(core_axis_name="c", subcore_axis_name="s")

    @functools.partial(
        pl.kernel,
        out_type=jax.ShapeDtypeStruct((NC * NP, EMB), jnp.float32),
        mesh=mesh,
        scratch_types=[
            pltpu.VMEM((KCH, CH), jnp.int32),
            pltpu.VMEM((KCH, CH), jnp.int32),
            pltpu.VMEM((KCH, CH), jnp.float32),
            pltpu.VMEM((CH, EMB), jnp.float32),
            pltpu.VMEM_SHARED((NP, EMB), jnp.float32),
            pltpu.SemaphoreType.DMA,
        ],
        compiler_params=pltpu.CompilerParams(use_tc_tiling_on_sc=False),
    )
    def k(table_hbm, src_hbm, dst_hbm, w_hbm, z_hbm, out_hbm,
          src_v, dst_v, w_v, rows_v, acc_sh, sem):
        c = lax.axis_index("c")
        s = lax.axis_index("s")

        pltpu.sync_copy(src_hbm.at[s], src_v)
        pltpu.sync_copy(dst_hbm.at[s], dst_v)
        pltpu.sync_copy(w_hbm.at[s], w_v)

        off = (c * N).astype(jnp.int32)

        def add_off(j, carry):
            for q in range(CH // L):
                sl = pl.ds(q * L, L)
                src_v[j, sl] = src_v[j, sl] + off
            return carry

        lax.fori_loop(0, KCH, add_off, 0)

        pltpu.sync_copy(z_hbm.at[pl.ds(s * RPT, RPT)],
                        acc_sh.at[pl.ds(s * RPT, RPT)])
        plsc.subcore_barrier()

        def chunk(j, carry):
            pltpu.async_copy(table_hbm.at[src_v.at[j]], rows_v, sem).wait()

            for g in range(CH // L):
                w16 = w_v[j, pl.ds(g * L, L)]
                for rr in range(L):
                    r = g * L + rr
                    wr = w16[rr]
                    for q in range(EMB // L):
                        sl = pl.ds(q * L, L)
                        rows_v[r, sl] = rows_v[r, sl] * wr

            pltpu.sync_copy(rows_v, acc_sh.at[dst_v.at[j]], add=True)
            return carry

        lax.fori_loop(0, KCH, chunk, 0)
        plsc.subcore_barrier()

        pltpu.sync_copy(acc_sh.at[pl.ds(s * RPT, RPT)],
                        out_hbm.at[pl.ds(c * NP + s * RPT, RPT)])

    return k(table2, src_t, dst_t, w_t, zeros)


_BM = 1000


def _dot(a, b):
    return lax.dot_general(a, b, (((1,), (0,)), ((), ())),
                           preferred_element_type=jnp.float32)


def _mm(h, w):
    m, kdim = h.shape

    def body(h_ref, w_ref, o_ref):
        o_ref[...] = _dot(h_ref[...], w_ref[...])

    return pl.pallas_call(
        body,
        grid=(m // _BM,),
        in_specs=[
            pl.BlockSpec((_BM, kdim), lambda i: (i, 0)),
            pl.BlockSpec((kdim, EMB), lambda i: (0, 0)),
        ],
        out_specs=pl.BlockSpec((_BM, EMB), lambda i: (i, 0)),
        out_shape=jax.ShapeDtypeStruct((m, EMB), jnp.float32),
    )(h, w)


def _elu(t):
    return jnp.where(t > 0, t, jnp.exp(t) - 1.0)


def _elu_mm(t, w):
    def body(t_ref, w_ref, h_ref, g_ref):
        h = _elu(t_ref[...])
        h_ref[...] = h
        g_ref[...] = _dot(h, w_ref[...])

    return pl.pallas_call(
        body,
        grid=(N // _BM,),
        in_specs=[
            pl.BlockSpec((_BM, EMB), lambda i: (i, 0)),
            pl.BlockSpec((EMB, EMB), lambda i: (0, 0)),
        ],
        out_specs=[
            pl.BlockSpec((_BM, EMB), lambda i: (i, 0)),
            pl.BlockSpec((_BM, EMB), lambda i: (i, 0)),
        ],
        out_shape=[
            jax.ShapeDtypeStruct((N, EMB), jnp.float32),
            jax.ShapeDtypeStruct((N, EMB), jnp.float32),
        ],
    )(t, w)


def _combine(h1a, h2a, t3a, h1b, h2b, t3b, a_layer, a_drug, a_dis, wd):
    assert NUM_R % _BM == 0

    def body(h1a_ref, h2a_ref, t3a_ref, h1b_ref, h2b_ref, t3b_ref,
             al_ref, ad_ref, as_ref, wd_ref, fin_ref, llm_ref, r_ref):
        i = pl.program_id(0)
        al = al_ref[...]
        wl = jnp.exp(al - jnp.max(al))
        wl = wl / jnp.sum(wl)
        e3a = _elu(t3a_ref[...])
        e3b = _elu(t3b_ref[...])
        emb = wl[0] * h1a_ref[...] + wl[1] * h2a_ref[...] + wl[2] * e3a
        lemb = wl[0] * h1b_ref[...] + wl[1] * h2b_ref[...] + wl[2] * e3b
        llm_ref[...] = lemb

        ad = ad_ref[...]
        wd2 = jnp.exp(ad - jnp.max(ad))
        wd2 = wd2 / jnp.sum(wd2)
        asv = as_ref[...]
        ws2 = jnp.exp(asv - jnp.max(asv))
        ws2 = ws2 / jnp.sum(ws2)

        is_drug = i < NUM_R // _BM
        w0 = jnp.where(is_drug, wd2[0], ws2[0])
        w1 = jnp.where(is_drug, wd2[1], ws2[1])
        fin = w0 * emb + w1 * lemb
        fin_ref[...] = fin
        r_ref[...] = _dot(fin, wd_ref[...])

    vspec = pl.BlockSpec((_BM, EMB), lambda i: (i, 0))
    return pl.pallas_call(
        body,
        grid=(N // _BM,),
        in_specs=[vspec] * 6 + [
            pl.BlockSpec((3,), lambda i: (0,)),
            pl.BlockSpec((2,), lambda i: (0,)),
            pl.BlockSpec((2,), lambda i: (0,)),
            pl.BlockSpec((EMB, EMB), lambda i: (0, 0)),
        ],
        out_specs=[vspec, vspec, vspec],
        out_shape=[
            jax.ShapeDtypeStruct((N, EMB), jnp.float32),
            jax.ShapeDtypeStruct((N, EMB), jnp.float32),
            jax.ShapeDtypeStruct((N, EMB), jnp.float32),
        ],
    )(h1a, h2a, t3a, h1b, h2b, t3b, a_layer, a_drug, a_dis, wd)


def _decoder(r, d):
    bm = 200

    def body(r_ref, d_ref, o_ref):
        acc = lax.dot_general(r_ref[...], d_ref[...],
                              (((1,), (1,)), ((), ())),
                              preferred_element_type=jnp.float32)
        o_ref[...] = jax.nn.sigmoid(acc)

    return pl.pallas_call(
        body,
        grid=(NUM_R // bm,),
        in_specs=[
            pl.BlockSpec((bm, EMB), lambda i: (i, 0)),
            pl.BlockSpec((NUM_R, EMB), lambda i: (0, 0)),
        ],
        out_specs=pl.BlockSpec((bm, NUM_R), lambda i: (i, 0)),
        out_shape=jax.ShapeDtypeStruct((NUM_R, NUM_R), jnp.float32),
    )(r, d)


def kernel(x, drug_emb, dis_emb, edge_index, edge_weight,
           W1, W2, W3, W4, W5, W6, a_layer, a_drug, a_dis, Wd):
    pad = EP - E
    src = jnp.concatenate(
        [edge_index[0].astype(jnp.int32), jnp.zeros((pad,), jnp.int32)])
    dst = jnp.concatenate(
        [edge_index[1].astype(jnp.int32), jnp.zeros((pad,), jnp.int32)])
    wgt = jnp.concatenate(
        [edge_weight.astype(jnp.float32), jnp.zeros((pad,), jnp.float32)])
    src_t = src.reshape(NS, KCH, CH)
    dst_t = dst.reshape(NS, KCH, CH)
    w_t = wgt.reshape(NS, KCH, CH)
    zeros = jnp.zeros((NP, EMB), jnp.float32)

    llm_x = jnp.concatenate([drug_emb, dis_emb], axis=0)

    g1a = _mm(x, W1)
    g1b = _mm(llm_x, W4)

    t1 = _spmm_sc(jnp.concatenate([g1a, g1b], axis=0), src_t, dst_t, w_t, zeros)
    h1a, g2a = _elu_mm(t1[:N], W2)
    h1b, g2b = _elu_mm(t1[NP:NP + N], W5)

    t2 = _spmm_sc(jnp.concatenate([g2a, g2b], axis=0), src_t, dst_t, w_t, zeros)
    h2a, g3a = _elu_mm(t2[:N], W3)
    h2b, g3b = _elu_mm(t2[NP:NP + N], W6)

    t3 = _spmm_sc(jnp.concatenate([g3a, g3b], axis=0), src_t, dst_t, w_t, zeros)

    final, llm_embeddings, r_full = _combine(
        h1a, h2a, t3[:N], h1b, h2b, t3[NP:NP + N], a_layer, a_drug, a_dis, Wd)

    recon = _decoder(r_full[:NUM_R], final[NUM_R:]).reshape(-1)
    return (recon, final, llm_embeddings)
